# baseline (speedup 1.0000x reference)
import functools

import jax
import jax.numpy as jnp
from jax import lax
from jax.experimental import pallas as pl
from jax.experimental.pallas import tpu as pltpu
from jax.experimental.pallas import tpu_sc as plsc

BATCH = 16384
DIM = 32
NC = 2
NS = 16
NW = NC * NS
BW = BATCH // NW
CH = 128
NCH = BW // CH
GROUPS = BW // 16


def _body(x0_hbm, x1_hbm, u_hbm, ub_hbm, p_hbm, pb_hbm, out_hbm,
          idx0, idx1, ur, pr, ubv, pbv, ov, sem):
    wid = lax.axis_index("s") * NC + lax.axis_index("c")

    pltpu.sync_copy(x0_hbm.at[wid], idx0)
    pltpu.sync_copy(x1_hbm.at[wid], idx1)

    copies = []
    for c in range(NCH):
        dst = pl.ds(c * CH, CH)
        copies.append(pltpu.async_copy(u_hbm.at[idx0.at[c]], ur.at[dst], sem))
        copies.append(pltpu.async_copy(p_hbm.at[idx1.at[c]], pr.at[dst], sem))
        copies.append(pltpu.async_copy(ub_hbm.at[idx0.at[c]], ubv.at[dst], sem))
        copies.append(pltpu.async_copy(pb_hbm.at[idx1.at[c]], pbv.at[dst], sem))
    for cp in copies:
        cp.wait()

    lane = lax.broadcasted_iota(jnp.int32, (16,), 0)

    def group(g, carry):
        base = g * 16
        rows = base + lane
        acc = ubv[pl.ds(base, 16)] + pbv[pl.ds(base, 16)]
        for d in range(DIM):
            cols = jnp.full((16,), d, jnp.int32)
            uu = plsc.load_gather(ur, [rows, cols])
            pp = plsc.load_gather(pr, [rows, cols])
            acc = acc + uu * pp
        ov[pl.ds(base, 16)] = 1.05 / (1.0 + jnp.exp(-acc))
        return carry

    lax.fori_loop(0, GROUPS, group, 0)
    pltpu.sync_copy(ov, out_hbm.at[pl.ds(wid * BW, BW)])


_sc_call = functools.partial(
    pl.kernel,
    mesh=plsc.VectorSubcoreMesh(
        core_axis_name="c", subcore_axis_name="s", num_cores=NC, num_subcores=NS
    ),
    out_type=jax.ShapeDtypeStruct((BATCH,), jnp.float32),
    compiler_params=pltpu.CompilerParams(
        needs_layout_passes=False, use_tc_tiling_on_sc=False
    ),
    scratch_types=[
        pltpu.VMEM((NCH, CH), jnp.int32),
        pltpu.VMEM((NCH, CH), jnp.int32),
        pltpu.VMEM((BW, DIM), jnp.float32),
        pltpu.VMEM((BW, DIM), jnp.float32),
        pltpu.VMEM((BW,), jnp.float32),
        pltpu.VMEM((BW,), jnp.float32),
        pltpu.VMEM((BW,), jnp.float32),
        pltpu.SemaphoreType.DMA,
    ],
)(_body)


def kernel(x, u, u_bias, p, p_bias):
    x0 = x[:, 0].astype(jnp.int32).reshape(NW, NCH, CH)
    x1 = x[:, 1].astype(jnp.int32).reshape(NW, NCH, CH)
    ub = u_bias.reshape(-1)
    pb = p_bias.reshape(-1)
    return _sc_call(x0, x1, u, ub, p, pb)
